# baseline (speedup 1.0000x reference)
import jax
import jax.numpy as jnp
from jax import lax
from jax.experimental import pallas as pl
from jax.experimental.pallas import tpu as pltpu
from jax.experimental.pallas import tpu_sc as plsc

_R = 128
_N = 100000
_NW = 32
_NG = 16
_RPW = _R // _NG
_CH = 6272
_NCH = 8
_HALF1 = 49792
_TAIL = 99968
_U = 4
_NV = _CH // (16 * _U)
_BIG = 2**30

_mesh = plsc.VectorSubcoreMesh(core_axis_name="c", subcore_axis_name="s")


def _lex_merge(val_a, idx_a, val_b, idx_b):
    upd = (val_b > val_a) | ((val_b == val_a) & (idx_b < idx_a))
    return jnp.where(upd, val_b, val_a), jnp.where(upd, idx_b, idx_a)


def _sc_body(probs_hbm, val_out, idx_out, buf0, buf1, tbuf, vres_ref,
             ires_ref, sem0, sem1, tsem):
    wid = lax.axis_index("c") * 16 + lax.axis_index("s")
    g = wid // 2
    col_base = (wid % 2) * _HALF1
    rbase = g * _RPW
    bufs = (buf0, buf1)
    sems = (sem0, sem1)
    lanes = lax.iota(jnp.int32, 16)

    def copy(c):
        return pltpu.make_async_copy(
            probs_hbm.at[pl.ds(rbase, _RPW), pl.ds(col_base + c * _CH, _CH)],
            bufs[c % 2],
            sems[c % 2],
        )

    copy(0).start()
    pltpu.make_async_copy(
        probs_hbm.at[pl.ds(rbase, _RPW), pl.ds(_TAIL, _N - _TAIL)], tbuf, tsem
    ).start()

    rstate = [None] * _RPW
    for c in range(_NCH):
        if c + 1 < _NCH:
            copy(c + 1).start()
        copy(c).wait()
        buf = bufs[c % 2]
        cbase = col_base + c * _CH

        for r in range(_RPW):

            def body(i, carry, buf=buf, r=r):
                out = []
                base = i * (_U * 16)
                for k in range(_U):
                    vk, itk = carry[2 * k], carry[2 * k + 1]
                    v = buf[r, pl.ds(base + k * 16, 16)]
                    m = v > vk
                    out.append(jnp.where(m, v, vk))
                    out.append(jnp.where(m, i, itk))
                return tuple(out)

            init = []
            for _ in range(_U):
                init.append(jnp.full((16,), -1.0, jnp.float32))
                init.append(jnp.zeros((16,), jnp.int32))
            accs = lax.fori_loop(0, _NV, body, tuple(init))

            cval = None
            cidx = None
            for k in range(_U):
                vk, itk = accs[2 * k], accs[2 * k + 1]
                ik = itk * (_U * 16) + (cbase + k * 16) + lanes
                if cval is None:
                    cval, cidx = vk, ik
                else:
                    cval, cidx = _lex_merge(cval, cidx, vk, ik)
            if rstate[r] is None:
                rstate[r] = (cval, cidx)
            else:
                rstate[r] = _lex_merge(*rstate[r], cval, cidx)

    pltpu.make_async_copy(
        probs_hbm.at[pl.ds(rbase, _RPW), pl.ds(_TAIL, _N - _TAIL)], tbuf, tsem
    ).wait()
    for r in range(_RPW):
        for k in range(2):
            v = tbuf[r, pl.ds(k * 16, 16)]
            rstate[r] = _lex_merge(*rstate[r], v, _TAIL + k * 16 + lanes)

    for r in range(_RPW):
        rv, ri = rstate[r]
        vres_ref[r, :] = rv
        ires_ref[r, :] = ri
    pltpu.make_async_copy(vres_ref, val_out.at[wid], sem0).start()
    pltpu.make_async_copy(vres_ref, val_out.at[wid], sem0).wait()
    pltpu.make_async_copy(ires_ref, idx_out.at[wid], sem1).start()
    pltpu.make_async_copy(ires_ref, idx_out.at[wid], sem1).wait()


def _merge_kernel(val_ref, idx_ref, out_ref):
    v = val_ref[...]
    ix = idx_ref[...]
    rm = jnp.max(v, axis=(1, 3), keepdims=True)
    cand = jnp.where(v == rm, ix, _BIG)
    out_ref[...] = jnp.min(cand, axis=(1, 3))


def kernel(probs):
    val_out, idx_out = pl.kernel(
        _sc_body,
        out_type=[
            jax.ShapeDtypeStruct((_NW, _RPW, 16), jnp.float32),
            jax.ShapeDtypeStruct((_NW, _RPW, 16), jnp.int32),
        ],
        mesh=_mesh,
        scratch_types=[
            pltpu.VMEM((_RPW, _CH), jnp.float32),
            pltpu.VMEM((_RPW, _CH), jnp.float32),
            pltpu.VMEM((_RPW, _N - _TAIL), jnp.float32),
            pltpu.VMEM((_RPW, 16), jnp.float32),
            pltpu.VMEM((_RPW, 16), jnp.int32),
            pltpu.SemaphoreType.DMA,
            pltpu.SemaphoreType.DMA,
            pltpu.SemaphoreType.DMA,
        ],
    )(probs)
    out = pl.pallas_call(
        _merge_kernel,
        out_shape=jax.ShapeDtypeStruct((_NG, _RPW), jnp.int32),
    )(
        val_out.reshape(_NG, 2, _RPW, 16),
        idx_out.reshape(_NG, 2, _RPW, 16),
    )
    return out.reshape(_R)


# speedup vs baseline: 1.0387x; 1.0387x over previous
import jax
import jax.numpy as jnp
from jax import lax
from jax.experimental import pallas as pl
from jax.experimental.pallas import tpu as pltpu
from jax.experimental.pallas import tpu_sc as plsc

_R = 128
_N = 100000
_NW = 32
_NG = 16
_RPW = _R // _NG
_CH = 6272
_NCH = 8
_HALF1 = 49792
_TAIL = 99968
_U = 8
_NV = _CH // (16 * _U)
_BIG = 2**30

_mesh = plsc.VectorSubcoreMesh(core_axis_name="c", subcore_axis_name="s")


def _lex_merge(val_a, idx_a, val_b, idx_b):
    upd = (val_b > val_a) | ((val_b == val_a) & (idx_b < idx_a))
    return jnp.where(upd, val_b, val_a), jnp.where(upd, idx_b, idx_a)


def _sc_body(probs_hbm, val_out, idx_out, buf0, buf1, tbuf, vres_ref,
             ires_ref, sem0, sem1, tsem):
    wid = lax.axis_index("c") * 16 + lax.axis_index("s")
    g = wid // 2
    col_base = (wid % 2) * _HALF1
    rbase = g * _RPW
    bufs = (buf0, buf1)
    sems = (sem0, sem1)
    lanes = lax.iota(jnp.int32, 16)

    def copy(c):
        return pltpu.make_async_copy(
            probs_hbm.at[pl.ds(rbase, _RPW), pl.ds(col_base + c * _CH, _CH)],
            bufs[c % 2],
            sems[c % 2],
        )

    copy(0).start()
    pltpu.make_async_copy(
        probs_hbm.at[pl.ds(rbase, _RPW), pl.ds(_TAIL, _N - _TAIL)], tbuf, tsem
    ).start()

    rstate = [None] * _RPW
    for c in range(_NCH):
        if c + 1 < _NCH:
            copy(c + 1).start()
        copy(c).wait()
        buf = bufs[c % 2]
        cbase = col_base + c * _CH

        for r in range(_RPW):

            def body(i, carry, buf=buf, r=r):
                out = []
                base = i * (_U * 16)
                for k in range(_U):
                    vk, itk = carry[2 * k], carry[2 * k + 1]
                    v = buf[r, pl.ds(base + k * 16, 16)]
                    m = v > vk
                    out.append(jnp.where(m, v, vk))
                    out.append(jnp.where(m, i, itk))
                return tuple(out)

            init = []
            for _ in range(_U):
                init.append(jnp.full((16,), -1.0, jnp.float32))
                init.append(jnp.zeros((16,), jnp.int32))
            accs = lax.fori_loop(0, _NV, body, tuple(init))

            cval = None
            cidx = None
            for k in range(_U):
                vk, itk = accs[2 * k], accs[2 * k + 1]
                ik = itk * (_U * 16) + (cbase + k * 16) + lanes
                if cval is None:
                    cval, cidx = vk, ik
                else:
                    cval, cidx = _lex_merge(cval, cidx, vk, ik)
            if rstate[r] is None:
                rstate[r] = (cval, cidx)
            else:
                rstate[r] = _lex_merge(*rstate[r], cval, cidx)

    pltpu.make_async_copy(
        probs_hbm.at[pl.ds(rbase, _RPW), pl.ds(_TAIL, _N - _TAIL)], tbuf, tsem
    ).wait()
    for r in range(_RPW):
        for k in range(2):
            v = tbuf[r, pl.ds(k * 16, 16)]
            rstate[r] = _lex_merge(*rstate[r], v, _TAIL + k * 16 + lanes)

    for r in range(_RPW):
        rv, ri = rstate[r]
        vres_ref[r, :] = rv
        ires_ref[r, :] = ri
    pltpu.make_async_copy(vres_ref, val_out.at[wid], sem0).start()
    pltpu.make_async_copy(vres_ref, val_out.at[wid], sem0).wait()
    pltpu.make_async_copy(ires_ref, idx_out.at[wid], sem1).start()
    pltpu.make_async_copy(ires_ref, idx_out.at[wid], sem1).wait()


def _merge_kernel(val_ref, idx_ref, out_ref):
    v = val_ref[...]
    ix = idx_ref[...]
    rm = jnp.max(v, axis=(1, 3), keepdims=True)
    cand = jnp.where(v == rm, ix, _BIG)
    out_ref[...] = jnp.min(cand, axis=(1, 3))


def kernel(probs):
    val_out, idx_out = pl.kernel(
        _sc_body,
        out_type=[
            jax.ShapeDtypeStruct((_NW, _RPW, 16), jnp.float32),
            jax.ShapeDtypeStruct((_NW, _RPW, 16), jnp.int32),
        ],
        mesh=_mesh,
        scratch_types=[
            pltpu.VMEM((_RPW, _CH), jnp.float32),
            pltpu.VMEM((_RPW, _CH), jnp.float32),
            pltpu.VMEM((_RPW, _N - _TAIL), jnp.float32),
            pltpu.VMEM((_RPW, 16), jnp.float32),
            pltpu.VMEM((_RPW, 16), jnp.int32),
            pltpu.SemaphoreType.DMA,
            pltpu.SemaphoreType.DMA,
            pltpu.SemaphoreType.DMA,
        ],
    )(probs)
    out = pl.pallas_call(
        _merge_kernel,
        out_shape=jax.ShapeDtypeStruct((_NG, _RPW), jnp.int32),
    )(
        val_out.reshape(_NG, 2, _RPW, 16),
        idx_out.reshape(_NG, 2, _RPW, 16),
    )
    return out.reshape(_R)


# speedup vs baseline: 1.1497x; 1.1069x over previous
import jax
import jax.numpy as jnp
from jax import lax
from jax.experimental import pallas as pl
from jax.experimental.pallas import tpu as pltpu
from jax.experimental.pallas import tpu_sc as plsc

_R = 128
_N = 100000
_NW = 32
_NG = 16
_RPW = _R // _NG
_CH = 6272
_NCH = 4
_SCBASE = 49792
_TAIL = 99968
_U = 8
_NV = _CH // (16 * _U)
_TCW = _SCBASE
_TRB = 16
_BIG = 2**30

_mesh = plsc.VectorSubcoreMesh(core_axis_name="c", subcore_axis_name="s")


def _lex_merge(val_a, idx_a, val_b, idx_b):
    upd = (val_b > val_a) | ((val_b == val_a) & (idx_b < idx_a))
    return jnp.where(upd, val_b, val_a), jnp.where(upd, idx_b, idx_a)


def _sc_body(probs_hbm, val_out, idx_out, buf0, buf1, tbuf, vres_ref,
             ires_ref, sem0, sem1, tsem):
    wid = lax.axis_index("c") * 16 + lax.axis_index("s")
    g = wid // 2
    col_base = _SCBASE + (wid % 2) * (_NCH * _CH)
    rbase = g * _RPW
    bufs = (buf0, buf1)
    sems = (sem0, sem1)
    lanes = lax.iota(jnp.int32, 16)

    def copy(c):
        return pltpu.make_async_copy(
            probs_hbm.at[pl.ds(rbase, _RPW), pl.ds(col_base + c * _CH, _CH)],
            bufs[c % 2],
            sems[c % 2],
        )

    copy(0).start()
    pltpu.make_async_copy(
        probs_hbm.at[pl.ds(rbase, _RPW), pl.ds(_TAIL, _N - _TAIL)], tbuf, tsem
    ).start()

    rstate = [None] * _RPW
    for c in range(_NCH):
        if c + 1 < _NCH:
            copy(c + 1).start()
        copy(c).wait()
        buf = bufs[c % 2]
        cbase = col_base + c * _CH

        for r in range(_RPW):

            def body(i, carry, buf=buf, r=r):
                out = []
                base = i * (_U * 16)
                for k in range(_U):
                    vk, itk = carry[2 * k], carry[2 * k + 1]
                    v = buf[r, pl.ds(base + k * 16, 16)]
                    m = v > vk
                    out.append(jnp.where(m, v, vk))
                    out.append(jnp.where(m, i, itk))
                return tuple(out)

            init = []
            for _ in range(_U):
                init.append(jnp.full((16,), -1.0, jnp.float32))
                init.append(jnp.zeros((16,), jnp.int32))
            accs = lax.fori_loop(0, _NV, body, tuple(init))

            cval = None
            cidx = None
            for k in range(_U):
                vk, itk = accs[2 * k], accs[2 * k + 1]
                ik = itk * (_U * 16) + (cbase + k * 16) + lanes
                if cval is None:
                    cval, cidx = vk, ik
                else:
                    cval, cidx = _lex_merge(cval, cidx, vk, ik)
            if rstate[r] is None:
                rstate[r] = (cval, cidx)
            else:
                rstate[r] = _lex_merge(*rstate[r], cval, cidx)

    pltpu.make_async_copy(
        probs_hbm.at[pl.ds(rbase, _RPW), pl.ds(_TAIL, _N - _TAIL)], tbuf, tsem
    ).wait()
    for r in range(_RPW):
        for k in range(2):
            v = tbuf[r, pl.ds(k * 16, 16)]
            rstate[r] = _lex_merge(*rstate[r], v, _TAIL + k * 16 + lanes)

    for r in range(_RPW):
        rv, ri = rstate[r]
        vres_ref[r, :] = rv
        ires_ref[r, :] = ri

    pltpu.make_async_copy(vres_ref, val_out.at[wid], sem0).start()
    pltpu.make_async_copy(vres_ref, val_out.at[wid], sem0).wait()
    pltpu.make_async_copy(ires_ref, idx_out.at[wid], sem1).start()
    pltpu.make_async_copy(ires_ref, idx_out.at[wid], sem1).wait()


def _tc_body(x_ref, val_ref, idx_ref):
    x = x_ref[...]
    val_ref[...] = jnp.max(x, axis=1, keepdims=True)
    idx_ref[...] = jnp.argmax(x, axis=1).astype(jnp.int32)[:, None]


def _merge_kernel(scval_ref, scidx_ref, tcval_ref, tcidx_ref, out_ref):
    v = scval_ref[...]
    ix = scidx_ref[...]
    rm = jnp.max(v, axis=(1, 3))
    cand = jnp.where(v == rm[:, None, :, None], ix, _BIG)
    ri = jnp.min(cand, axis=(1, 3))
    tv = tcval_ref[...]
    ti = tcidx_ref[...]
    upd = rm > tv
    out_ref[...] = jnp.where(upd, ri, ti)


def kernel(probs):
    val_out, idx_out = pl.kernel(
        _sc_body,
        out_type=[
            jax.ShapeDtypeStruct((_NW, _RPW, 16), jnp.float32),
            jax.ShapeDtypeStruct((_NW, _RPW, 16), jnp.int32),
        ],
        mesh=_mesh,
        scratch_types=[
            pltpu.VMEM((_RPW, _CH), jnp.float32),
            pltpu.VMEM((_RPW, _CH), jnp.float32),
            pltpu.VMEM((_RPW, _N - _TAIL), jnp.float32),
            pltpu.VMEM((_RPW, 16), jnp.float32),
            pltpu.VMEM((_RPW, 16), jnp.int32),
            pltpu.SemaphoreType.DMA,
            pltpu.SemaphoreType.DMA,
            pltpu.SemaphoreType.DMA,
        ],
    )(probs)

    tcval, tcidx = pl.pallas_call(
        _tc_body,
        grid=(_R // _TRB,),
        in_specs=[pl.BlockSpec((_TRB, _TCW), lambda i: (i, 0))],
        out_specs=[
            pl.BlockSpec((_TRB, 1), lambda i: (i, 0)),
            pl.BlockSpec((_TRB, 1), lambda i: (i, 0)),
        ],
        out_shape=[
            jax.ShapeDtypeStruct((_R, 1), jnp.float32),
            jax.ShapeDtypeStruct((_R, 1), jnp.int32),
        ],
    )(probs)

    out = pl.pallas_call(
        _merge_kernel,
        out_shape=jax.ShapeDtypeStruct((_NG, _RPW), jnp.int32),
    )(
        val_out.reshape(_NG, 2, _RPW, 16),
        idx_out.reshape(_NG, 2, _RPW, 16),
        tcval.reshape(_NG, _RPW),
        tcidx.reshape(_NG, _RPW),
    )
    return out.reshape(_R)
